# baseline (speedup 1.0000x reference)
import functools

import jax
import jax.numpy as jnp
from jax import lax
from jax.experimental import pallas as pl
from jax.experimental.pallas import tpu as pltpu
from jax.experimental.pallas import tpu_sc as plsc

N_NODES = 10000
N_EDGES = 320000
D = 128

_NC = 2
_NS = 16
_NW = _NC * _NS
_EPW = N_EDGES // _NW
_CH = 128
_FULL = _EPW // _CH
_TAIL = _EPW - _FULL * _CH


def _ln(x):
    mu = jnp.mean(x, axis=-1, keepdims=True)
    var = jnp.mean((x - mu) ** 2, axis=-1, keepdims=True)
    return (x - mu) / jnp.sqrt(var + 1e-6)


def _enc_node_body(x_ref, w1_ref, b1_ref, w2_ref, b2_ref, o_ref):
    x = x_ref[...]
    t = jnp.maximum(jnp.dot(x, w1_ref[...], preferred_element_type=jnp.float32)
                    + b1_ref[...], 0.0)
    y = jnp.dot(t, w2_ref[...], preferred_element_type=jnp.float32) + b2_ref[...]
    o_ref[...] = _ln(y)


def _enc_apply(x, p, block_rows):
    n, din = x.shape
    w1, b1 = p[0]["w"], p[0]["b"].reshape(1, -1)
    w2, b2 = p[1]["w"], p[1]["b"].reshape(1, -1)
    grid = (n // block_rows,)
    return pl.pallas_call(
        _enc_node_body,
        grid=grid,
        in_specs=[
            pl.BlockSpec((block_rows, din), lambda i: (i, 0)),
            pl.BlockSpec(w1.shape, lambda i: (0, 0)),
            pl.BlockSpec(b1.shape, lambda i: (0, 0)),
            pl.BlockSpec(w2.shape, lambda i: (0, 0)),
            pl.BlockSpec(b2.shape, lambda i: (0, 0)),
        ],
        out_specs=pl.BlockSpec((block_rows, D), lambda i: (i, 0)),
        out_shape=jax.ShapeDtypeStruct((n, D), jnp.float32),
    )(x, w1, b1, w2, b2)


def _prep_body(h_ref, ws_ref, wr_ref, a_ref, b_ref):
    h = h_ref[...]
    a_ref[...] = jnp.dot(h, ws_ref[...], preferred_element_type=jnp.float32)
    b_ref[...] = jnp.dot(h, wr_ref[...], preferred_element_type=jnp.float32)


def _prep_tables(h, ws, wr, block_rows=2000):
    grid = (N_NODES // block_rows,)
    return pl.pallas_call(
        _prep_body,
        grid=grid,
        in_specs=[
            pl.BlockSpec((block_rows, D), lambda i: (i, 0)),
            pl.BlockSpec((D, D), lambda i: (0, 0)),
            pl.BlockSpec((D, D), lambda i: (0, 0)),
        ],
        out_specs=[
            pl.BlockSpec((block_rows, D), lambda i: (i, 0)),
            pl.BlockSpec((block_rows, D), lambda i: (i, 0)),
        ],
        out_shape=[
            jax.ShapeDtypeStruct((N_NODES, D), jnp.float32),
            jax.ShapeDtypeStruct((N_NODES, D), jnp.float32),
        ],
    )(h, ws, wr)


def _edge_body(hs_ref, hr_ref, ee_ref, we_ref, b1_ref, w2_ref, b2_ref, o_ref):
    br = o_ref.shape[0]
    hs = hs_ref[...].reshape(br, D)
    hr = hr_ref[...].reshape(br, D)
    pre = (hs + hr
           + jnp.dot(ee_ref[...], we_ref[...], preferred_element_type=jnp.float32)
           + b1_ref[...])
    t = jnp.maximum(pre, 0.0)
    y = jnp.dot(t, w2_ref[...], preferred_element_type=jnp.float32) + b2_ref[...]
    o_ref[...] = _ln(y)


_EBC = 20


def _edge_mlp(p4, ee, we, b1, w2, b2):
    block_rows = _EBC * _GCH
    grid = (N_EDGES // block_rows,)
    b1 = b1.reshape(1, -1)
    b2 = b2.reshape(1, -1)
    return pl.pallas_call(
        _edge_body,
        grid=grid,
        in_specs=[
            pl.BlockSpec((_EBC, 1, _GCH, D), lambda i: (i, 0, 0, 0)),
            pl.BlockSpec((_EBC, 1, _GCH, D), lambda i: (i, 1, 0, 0)),
            pl.BlockSpec((block_rows, D), lambda i: (i, 0)),
            pl.BlockSpec((D, D), lambda i: (0, 0)),
            pl.BlockSpec((1, D), lambda i: (0, 0)),
            pl.BlockSpec((D, D), lambda i: (0, 0)),
            pl.BlockSpec((1, D), lambda i: (0, 0)),
        ],
        out_specs=pl.BlockSpec((block_rows, D), lambda i: (i, 0)),
        out_shape=jax.ShapeDtypeStruct((N_EDGES, D), jnp.float32),
    )(p4, p4, ee, we, b1, w2, b2)


def _node_body(h_ref, p0_ref, p1_ref, u1h_ref, u1a_ref, b1_ref, u2_ref, b2_ref,
               o_ref):
    h = h_ref[...]
    agg = p0_ref[...] + p1_ref[...]
    t = jnp.maximum(
        jnp.dot(h, u1h_ref[...], preferred_element_type=jnp.float32)
        + jnp.dot(agg, u1a_ref[...], preferred_element_type=jnp.float32)
        + b1_ref[...], 0.0)
    y = jnp.dot(t, u2_ref[...], preferred_element_type=jnp.float32) + b2_ref[...]
    o_ref[...] = h + _ln(y)


def _node_mlp(h, p0, p1, u1h, u1a, b1, u2, b2, block_rows=2000):
    grid = (N_NODES // block_rows,)
    b1 = b1.reshape(1, -1)
    b2 = b2.reshape(1, -1)
    return pl.pallas_call(
        _node_body,
        grid=grid,
        in_specs=[
            pl.BlockSpec((block_rows, D), lambda i: (i, 0)),
            pl.BlockSpec((block_rows, D), lambda i: (i, 0)),
            pl.BlockSpec((block_rows, D), lambda i: (i, 0)),
            pl.BlockSpec((D, D), lambda i: (0, 0)),
            pl.BlockSpec((D, D), lambda i: (0, 0)),
            pl.BlockSpec((1, D), lambda i: (0, 0)),
            pl.BlockSpec((D, D), lambda i: (0, 0)),
            pl.BlockSpec((1, D), lambda i: (0, 0)),
        ],
        out_specs=pl.BlockSpec((block_rows, D), lambda i: (i, 0)),
        out_shape=jax.ShapeDtypeStruct((N_NODES, D), jnp.float32),
    )(h, p0, p1, u1h, u1a, b1, u2, b2)


_sc_mesh = plsc.VectorSubcoreMesh(core_axis_name="c", subcore_axis_name="s")


_GCH = 128
_SCH = 64
_NCHUNK_PAD = ((-(-N_EDGES // _GCH) + _NW - 1) // _NW) * _NW
_E_PAD = _NCHUNK_PAD * _GCH


def _sc_gather(a_tab, b_tab, s2d, r2d):
    @functools.partial(
        pl.kernel, mesh=_sc_mesh,
        out_type=jax.ShapeDtypeStruct((_NCHUNK_PAD * 2 * _GCH, D), jnp.float32),
        scratch_types=[
            pltpu.SemaphoreType.DMA,
            pltpu.SemaphoreType.DMA,
        ],
    )
    def k(a_hbm, b_hbm, s_hbm, r_hbm, pg_hbm, sema, semb):
        def body(s_vmem, r_vmem, pg_vmem):
            cpa = pltpu.async_copy(a_hbm.at[s_vmem.at[0]],
                                   pg_vmem.at[pl.ds(0, _GCH)], sema)
            cpb = pltpu.async_copy(b_hbm.at[r_vmem.at[0]],
                                   pg_vmem.at[pl.ds(_GCH, _GCH)], semb)
            cpa.wait()
            cpb.wait()

        buf4 = pl.Buffered(buffer_count=4)
        pltpu.emit_pipeline(
            body,
            grid=(_NCHUNK_PAD,),
            in_specs=[
                pl.BlockSpec((1, _GCH), index_map=lambda i: (i, 0),
                             pipeline_mode=buf4),
                pl.BlockSpec((1, _GCH), index_map=lambda i: (i, 0),
                             pipeline_mode=buf4),
            ],
            out_specs=[
                pl.BlockSpec((2 * _GCH, D), index_map=lambda i: (i, 0)),
            ],
            core_axis_name=("c", "s"),
            dimension_semantics=(pltpu.PARALLEL,),
        )(s_hbm, r_hbm, pg_hbm)

    return k(a_tab, b_tab, s2d, r2d)


def _sc_scatter_add(msgs, receivers):
    n_chunks = N_EDGES // _SCH
    n_main = (n_chunks // _NW) * _NW
    _ZR = 16

    @functools.partial(
        pl.kernel, mesh=_sc_mesh,
        out_type=jax.ShapeDtypeStruct((_NC, N_NODES, D), jnp.float32),
        scratch_types=[
            pltpu.VMEM((_SCH,), jnp.int32),
            pltpu.VMEM((_SCH, D), jnp.float32),
            pltpu.VMEM((_ZR, D), jnp.float32),
            pltpu.VMEM_SHARED((N_NODES, D), jnp.float32),
        ],
    )
    def k(m_hbm, r_hbm, out_hbm, ridx_t, mbuf_t, zbuf, acc):
        c = lax.axis_index("c")
        s = lax.axis_index("s")
        wid = s * _NC + c

        @pl.loop(0, _ZR)
        def _(i):
            @pl.loop(0, D, step=16)
            def _(q):
                zbuf.at[pl.ds(i, 1), pl.ds(q, 16)][...] = (
                    jnp.zeros((1, 16), jnp.float32))

        n_zchunks = N_NODES // _ZR
        @pl.loop(0, (n_zchunks + _NS - 1) // _NS)
        def _(j):
            g = s + j * _NS
            @pl.when(g < n_zchunks)
            def _():
                pltpu.sync_copy(zbuf, acc.at[pl.ds(g * _ZR, _ZR)])

        plsc.subcore_barrier()

        def body(r_vmem, m_vmem):
            pltpu.sync_copy(m_vmem, acc.at[r_vmem.at[0]], add=True)

        pltpu.emit_pipeline(
            body,
            grid=(n_main,),
            in_specs=[
                pl.BlockSpec((1, _SCH), index_map=lambda i: (i, 0)),
                pl.BlockSpec((_SCH, D), index_map=lambda i: (i, 0)),
            ],
            out_specs=[],
            core_axis_name=("c", "s"),
            dimension_semantics=(pltpu.PARALLEL,),
        )(r_hbm, m_hbm)

        @pl.when(wid < n_chunks - n_main)
        def _():
            tc = n_main + wid
            pltpu.sync_copy(r_hbm.at[tc], ridx_t)
            pltpu.sync_copy(m_hbm.at[pl.ds(tc * _SCH, _SCH)], mbuf_t)
            pltpu.sync_copy(mbuf_t, acc.at[ridx_t], add=True)

        plsc.subcore_barrier()

        @pl.loop(0, (n_zchunks + _NS - 1) // _NS)
        def _(j):
            g = s + j * _NS
            @pl.when(g < n_zchunks)
            def _():
                pltpu.sync_copy(acc.at[pl.ds(g * _ZR, _ZR)],
                                out_hbm.at[c].at[pl.ds(g * _ZR, _ZR)])

    return k(msgs, receivers)


def kernel(nodes, edges, senders, receivers, params):
    senders = senders.astype(jnp.int32)
    receivers = receivers.astype(jnp.int32)
    pad = _E_PAD - N_EDGES
    s2d = jnp.pad(senders, (0, pad)).reshape(_NCHUNK_PAD, _GCH)
    r2d_pad = jnp.pad(receivers, (0, pad)).reshape(_NCHUNK_PAD, _GCH)
    r2d_s = receivers.reshape(N_EDGES // _SCH, _SCH)

    h = _enc_apply(nodes, params["enc_node"], block_rows=2000)
    ee = _enc_apply(edges, params["enc_edge"], block_rows=4000)

    for lp in params["layers"]:
        mw1 = lp["msg"][0]["w"]
        mb1 = lp["msg"][0]["b"]
        mw2, mb2 = lp["msg"][1]["w"], lp["msg"][1]["b"]
        ws, wr, we = mw1[:D], mw1[D:2 * D], mw1[2 * D:]

        a_tab, b_tab = _prep_tables(h, ws, wr)

        pg = _sc_gather(a_tab, b_tab, s2d, r2d_pad)
        p4 = pg.reshape(_NCHUNK_PAD, 2, _GCH, D)

        msgs = _edge_mlp(p4, ee, we, mb1, mw2, mb2)

        partials = _sc_scatter_add(msgs, r2d_s)

        nw1 = lp["node"][0]["w"]
        nb1 = lp["node"][0]["b"]
        nw2, nb2 = lp["node"][1]["w"], lp["node"][1]["b"]
        h = _node_mlp(h, partials[0], partials[1], nw1[:D], nw1[D:],
                      nb1, nw2, nb2)

    return h


# speedup vs baseline: 1.6814x; 1.6814x over previous
import functools

import jax
import jax.numpy as jnp
from jax import lax
from jax.experimental import pallas as pl
from jax.experimental.pallas import tpu as pltpu
from jax.experimental.pallas import tpu_sc as plsc

N_NODES = 10000
N_EDGES = 320000
D = 128

_NC = 2
_NS = 16
_NW = _NC * _NS

_GCH = 64
_SCH = 64
_NCHUNK_PAD = ((-(-N_EDGES // _GCH) + _NW - 1) // _NW) * _NW
_E_PAD = _NCHUNK_PAD * _GCH
_SPLIT_C = 2560
_EBR = 2560


def _ln(x):
    mu = jnp.mean(x, axis=-1, keepdims=True)
    var = jnp.mean((x - mu) ** 2, axis=-1, keepdims=True)
    return (x - mu) / jnp.sqrt(var + 1e-6)


def _enc_body(x_ref, w1_ref, b1_ref, w2_ref, b2_ref, o_ref):
    x = x_ref[...]
    t = jnp.maximum(jnp.dot(x, w1_ref[...], preferred_element_type=jnp.float32)
                    + b1_ref[...], 0.0)
    y = jnp.dot(t, w2_ref[...], preferred_element_type=jnp.float32) + b2_ref[...]
    o_ref[...] = _ln(y)


def _enc_apply(x, p, block_rows):
    n, din = x.shape
    w1, b1 = p[0]["w"], p[0]["b"].reshape(1, -1)
    w2, b2 = p[1]["w"], p[1]["b"].reshape(1, -1)
    grid = (n // block_rows,)
    return pl.pallas_call(
        _enc_body,
        grid=grid,
        in_specs=[
            pl.BlockSpec((block_rows, din), lambda i: (i, 0)),
            pl.BlockSpec(w1.shape, lambda i: (0, 0)),
            pl.BlockSpec(b1.shape, lambda i: (0, 0)),
            pl.BlockSpec(w2.shape, lambda i: (0, 0)),
            pl.BlockSpec(b2.shape, lambda i: (0, 0)),
        ],
        out_specs=pl.BlockSpec((block_rows, D), lambda i: (i, 0)),
        out_shape=jax.ShapeDtypeStruct((n, D), jnp.float32),
    )(x, w1, b1, w2, b2)


def _prep_body(h_ref, ws_ref, wr_ref, a_ref, b_ref):
    h = h_ref[...]
    a_ref[...] = jnp.dot(h, ws_ref[...], preferred_element_type=jnp.float32)
    b_ref[...] = jnp.dot(h, wr_ref[...], preferred_element_type=jnp.float32)


def _prep_tables(h, ws, wr, block_rows=2000):
    grid = (N_NODES // block_rows,)
    return pl.pallas_call(
        _prep_body,
        grid=grid,
        in_specs=[
            pl.BlockSpec((block_rows, D), lambda i: (i, 0)),
            pl.BlockSpec((D, D), lambda i: (0, 0)),
            pl.BlockSpec((D, D), lambda i: (0, 0)),
        ],
        out_specs=[
            pl.BlockSpec((block_rows, D), lambda i: (i, 0)),
            pl.BlockSpec((block_rows, D), lambda i: (i, 0)),
        ],
        out_shape=[
            jax.ShapeDtypeStruct((N_NODES, D), jnp.float32),
            jax.ShapeDtypeStruct((N_NODES, D), jnp.float32),
        ],
    )(h, ws, wr)


def _edge_body(hs_ref, hr_ref, ee_ref, we_ref, b1_ref, w2_ref, b2_ref, o_ref):
    pre = (hs_ref[...] + hr_ref[...]
           + jnp.dot(ee_ref[...], we_ref[...], preferred_element_type=jnp.float32)
           + b1_ref[...])
    t = jnp.maximum(pre, 0.0)
    y = jnp.dot(t, w2_ref[...], preferred_element_type=jnp.float32) + b2_ref[...]
    o_ref[...] = _ln(y)


def _edge_mlp(hs, hr, ee, we, b1, b2, w2, n_blocks, blk_off):
    return pl.pallas_call(
        _edge_body,
        grid=(n_blocks,),
        in_specs=[
            pl.BlockSpec((_EBR, D), lambda i: (i, 0)),
            pl.BlockSpec((_EBR, D), lambda i: (i, 0)),
            pl.BlockSpec((_EBR, D), lambda i: (i + blk_off, 0)),
            pl.BlockSpec((D, D), lambda i: (0, 0)),
            pl.BlockSpec((1, D), lambda i: (0, 0)),
            pl.BlockSpec((D, D), lambda i: (0, 0)),
            pl.BlockSpec((1, D), lambda i: (0, 0)),
        ],
        out_specs=pl.BlockSpec((_EBR, D), lambda i: (i, 0)),
        out_shape=jax.ShapeDtypeStruct((n_blocks * _EBR, D), jnp.float32),
    )(hs, hr, ee, we, b1.reshape(1, -1), w2, b2.reshape(1, -1))


def _node_body(h_ref, p0_ref, p1_ref, p2_ref, p3_ref,
               u1h_ref, u1a_ref, b1_ref, u2_ref, b2_ref, o_ref):
    h = h_ref[...]
    agg = (p0_ref[...] + p1_ref[...]) + (p2_ref[...] + p3_ref[...])
    t = jnp.maximum(
        jnp.dot(h, u1h_ref[...], preferred_element_type=jnp.float32)
        + jnp.dot(agg, u1a_ref[...], preferred_element_type=jnp.float32)
        + b1_ref[...], 0.0)
    y = jnp.dot(t, u2_ref[...], preferred_element_type=jnp.float32) + b2_ref[...]
    o_ref[...] = h + _ln(y)


def _node_mlp(h, parts, u1h, u1a, b1, u2, b2, block_rows=2000):
    grid = (N_NODES // block_rows,)
    row_spec = pl.BlockSpec((block_rows, D), lambda i: (i, 0))
    mat_spec = pl.BlockSpec((D, D), lambda i: (0, 0))
    vec_spec = pl.BlockSpec((1, D), lambda i: (0, 0))
    return pl.pallas_call(
        _node_body,
        grid=grid,
        in_specs=[row_spec, row_spec, row_spec, row_spec, row_spec,
                  mat_spec, mat_spec, vec_spec, mat_spec, vec_spec],
        out_specs=row_spec,
        out_shape=jax.ShapeDtypeStruct((N_NODES, D), jnp.float32),
    )(h, *parts, u1h, u1a, b1.reshape(1, -1), u2, b2.reshape(1, -1))


@functools.cache
def _sc_mesh_get():
    return plsc.VectorSubcoreMesh(core_axis_name="c", subcore_axis_name="s")


def _sc_gather(a_tab, b_tab, s2d, r2d, c0, c1):
    n = c1 - c0

    @functools.partial(
        pl.kernel, mesh=_sc_mesh_get(),
        out_type=[jax.ShapeDtypeStruct((n * _GCH, D), jnp.float32),
                  jax.ShapeDtypeStruct((n * _GCH, D), jnp.float32)],
        scratch_types=[
            pltpu.SemaphoreType.DMA,
            pltpu.SemaphoreType.DMA,
        ],
    )
    def k(a_hbm, b_hbm, s_hbm, r_hbm, hs_hbm, hr_hbm, sema, semb):
        def body(s_vmem, r_vmem, hs_vmem, hr_vmem):
            cpa = pltpu.async_copy(a_hbm.at[s_vmem.at[0]], hs_vmem, sema)
            cpb = pltpu.async_copy(b_hbm.at[r_vmem.at[0]], hr_vmem, semb)
            cpa.wait()
            cpb.wait()

        buf4 = pl.Buffered(buffer_count=4)
        pltpu.emit_pipeline(
            body,
            grid=(n,),
            in_specs=[
                pl.BlockSpec((1, _GCH), index_map=lambda i: (i + c0, 0),
                             pipeline_mode=buf4),
                pl.BlockSpec((1, _GCH), index_map=lambda i: (i + c0, 0),
                             pipeline_mode=buf4),
            ],
            out_specs=[
                pl.BlockSpec((_GCH, D), index_map=lambda i: (i, 0)),
                pl.BlockSpec((_GCH, D), index_map=lambda i: (i, 0)),
            ],
            core_axis_name=("c", "s"),
            dimension_semantics=(pltpu.PARALLEL,),
        )(s_hbm, r_hbm, hs_hbm, hr_hbm)

    return k(a_tab, b_tab, s2d, r2d)


def _sc_scatter_add(msgs, r2d_s, c0):
    n_chunks = msgs.shape[0] // _SCH
    n_main = (n_chunks // _NW) * _NW
    _ZR = 16

    @functools.partial(
        pl.kernel, mesh=_sc_mesh_get(),
        out_type=jax.ShapeDtypeStruct((_NC, N_NODES, D), jnp.float32),
        scratch_types=[
            pltpu.VMEM((_SCH,), jnp.int32),
            pltpu.VMEM((_SCH, D), jnp.float32),
            pltpu.VMEM((_ZR, D), jnp.float32),
            pltpu.VMEM_SHARED((N_NODES, D), jnp.float32),
        ],
    )
    def k(m_hbm, r_hbm, out_hbm, ridx_t, mbuf_t, zbuf, acc):
        c = lax.axis_index("c")
        s = lax.axis_index("s")
        wid = s * _NC + c

        @pl.loop(0, _ZR)
        def _(i):
            @pl.loop(0, D, step=16)
            def _(q):
                zbuf.at[pl.ds(i, 1), pl.ds(q, 16)][...] = (
                    jnp.zeros((1, 16), jnp.float32))

        n_zchunks = N_NODES // _ZR
        @pl.loop(0, (n_zchunks + _NS - 1) // _NS)
        def _(j):
            g = s + j * _NS
            @pl.when(g < n_zchunks)
            def _():
                pltpu.sync_copy(zbuf, acc.at[pl.ds(g * _ZR, _ZR)])

        plsc.subcore_barrier()

        def body(r_vmem, m_vmem):
            pltpu.sync_copy(m_vmem, acc.at[r_vmem.at[0]], add=True)

        pltpu.emit_pipeline(
            body,
            grid=(n_main,),
            in_specs=[
                pl.BlockSpec((1, _SCH), index_map=lambda i: (i + c0, 0)),
                pl.BlockSpec((_SCH, D), index_map=lambda i: (i, 0)),
            ],
            out_specs=[],
            core_axis_name=("c", "s"),
            dimension_semantics=(pltpu.PARALLEL,),
        )(r_hbm, m_hbm)

        if n_chunks > n_main:
            @pl.when(wid < n_chunks - n_main)
            def _():
                tc = n_main + wid
                pltpu.sync_copy(r_hbm.at[tc + c0], ridx_t)
                pltpu.sync_copy(m_hbm.at[pl.ds(tc * _SCH, _SCH)], mbuf_t)
                pltpu.sync_copy(mbuf_t, acc.at[ridx_t], add=True)

        plsc.subcore_barrier()

        @pl.loop(0, (n_zchunks + _NS - 1) // _NS)
        def _(j):
            g = s + j * _NS
            @pl.when(g < n_zchunks)
            def _():
                pltpu.sync_copy(acc.at[pl.ds(g * _ZR, _ZR)],
                                out_hbm.at[c].at[pl.ds(g * _ZR, _ZR)])

    return k(msgs, r2d_s)


def kernel(nodes, edges, senders, receivers, params):
    senders = senders.astype(jnp.int32)
    receivers = receivers.astype(jnp.int32)
    pad = _E_PAD - N_EDGES
    s2d = jnp.pad(senders, (0, pad)).reshape(_NCHUNK_PAD, _GCH)
    r2d = jnp.pad(receivers, (0, pad)).reshape(_NCHUNK_PAD, _GCH)
    r2d_s = receivers.reshape(N_EDGES // _SCH, _SCH)

    h = _enc_apply(nodes, params["enc_node"], block_rows=2000)
    ee = _enc_apply(edges, params["enc_edge"], block_rows=4000)

    split_e = _SPLIT_C * _GCH
    blocks0 = split_e // _EBR
    blocks1 = (N_EDGES - split_e) // _EBR

    for lp in params["layers"]:
        mw1 = lp["msg"][0]["w"]
        mb1 = lp["msg"][0]["b"]
        mw2, mb2 = lp["msg"][1]["w"], lp["msg"][1]["b"]
        ws, wr, we = mw1[:D], mw1[D:2 * D], mw1[2 * D:]

        a_tab, b_tab = _prep_tables(h, ws, wr)

        hs0, hr0 = _sc_gather(a_tab, b_tab, s2d, r2d, 0, _SPLIT_C)
        hs1, hr1 = _sc_gather(a_tab, b_tab, s2d, r2d, _SPLIT_C, _NCHUNK_PAD)

        msgs0 = _edge_mlp(hs0, hr0, ee, we, mb1, mb2, mw2, blocks0, 0)
        msgs1 = _edge_mlp(hs1, hr1, ee, we, mb1, mb2, mw2, blocks1, blocks0)

        part0 = _sc_scatter_add(msgs0, r2d_s, 0)
        part1 = _sc_scatter_add(msgs1, r2d_s, split_e // _SCH)

        nw1 = lp["node"][0]["w"]
        nb1 = lp["node"][0]["b"]
        nw2, nb2 = lp["node"][1]["w"], lp["node"][1]["b"]
        h = _node_mlp(h, (part0[0], part0[1], part1[0], part1[1]),
                      nw1[:D], nw1[D:], nb1, nw2, nb2)

    return h
